# Initial kernel scaffold; baseline (speedup 1.0000x reference)
import jax
import jax.numpy as jnp
from jax.experimental import pallas as pl


def kernel(tokens, tok_emb, norm_w, w_out, l0_attn_norm, l0_ffn_norm, l0_wq, l0_wk, l0_wv, l0_wo, l0_w1, l0_w3, l0_w2, l1_attn_norm, l1_ffn_norm, l1_wq, l1_wk, l1_wv, l1_wo, l1_w1, l1_w3, l1_w2):
    raise NotImplementedError("write your pallas kernel here")


# speedup vs baseline: 2.4198x; 2.4198x over previous
import functools
import math

import jax
import jax.numpy as jnp
from jax.experimental import pallas as pl
from jax.experimental.pallas import tpu as pltpu

_D = 2048
_HD = 128
_NH = 16
_NKV = 4
_REP = _NH // _NKV
_H = 7168
_V = 32000
_B = 4
_L = 512
_M = _B * _L
_EPS = 1e-5
_THETA = 10000.0

_TM = 1024
_TMW = 512
_TMF = 512
_TH = 512
_TV = 640
_TQ = 256


def _rmsnorm(x, nw):
    var = jnp.mean(x * x, axis=-1, keepdims=True)
    return (x * jax.lax.rsqrt(var + _EPS)) * nw


def _rope(y, cos, sin2):
    even = jax.lax.broadcasted_iota(jnp.int32, y.shape, 1) % 2 == 0
    n = y.shape[1]
    swap = jnp.where(even, pltpu.roll(y, n - 1, 1), pltpu.roll(y, 1, 1))
    return y * cos + swap * sin2


def _qkv_kernel(x_ref, nw_ref, wq_ref, wk_ref, wv_ref, cos_ref, sin_ref,
                o_ref, xn_ref):
    j = pl.program_id(1)

    @pl.when(j == 0)
    def _():
        xn_ref[...] = _rmsnorm(x_ref[...], nw_ref[...])

    @pl.when(j < 4)
    def _():
        y = jnp.dot(xn_ref[...], wq_ref[...],
                    preferred_element_type=jnp.float32)
        o_ref[...] = _rope(y, cos_ref[...], sin_ref[...])

    @pl.when(j == 4)
    def _():
        y = jnp.dot(xn_ref[...], wk_ref[...],
                    preferred_element_type=jnp.float32)
        o_ref[...] = _rope(y, cos_ref[...], sin_ref[...])

    @pl.when(j == 5)
    def _():
        o_ref[...] = jnp.dot(xn_ref[...], wv_ref[...],
                             preferred_element_type=jnp.float32)


def _qkv_rope(x, nw, wq, wk, wv, cos_t, sin_t):
    return pl.pallas_call(
        _qkv_kernel,
        grid=(_M // _TM, 6),
        in_specs=[
            pl.BlockSpec((_TM, _D), lambda i, j: (i, 0)),
            pl.BlockSpec((1, _D), lambda i, j: (0, 0)),
            pl.BlockSpec((_D, 512), lambda i, j: (0, jnp.minimum(j, 3))),
            pl.BlockSpec((_D, 512), lambda i, j: (0, 0)),
            pl.BlockSpec((_D, 512), lambda i, j: (0, 0)),
            pl.BlockSpec((_TM, 512), lambda i, j: (i, 0)),
            pl.BlockSpec((_TM, 512), lambda i, j: (i, 0)),
        ],
        out_specs=pl.BlockSpec((_TM, 512), lambda i, j: (i, j)),
        out_shape=jax.ShapeDtypeStruct((_M, _NH * _HD + 2 * _NKV * _HD),
                                       jnp.float32),
        scratch_shapes=[pltpu.VMEM((_TM, _D), jnp.float32)],
        compiler_params=pltpu.CompilerParams(
            dimension_semantics=("parallel", "arbitrary"),
            vmem_limit_bytes=56 * 1024 * 1024),
    )(x, nw.reshape(1, _D), wq, wk, wv, cos_t, sin_t)


def _attn_kernel(scale, q_ref, k_ref, v_ref, o_ref):
    q = q_ref[...]
    k = k_ref[...]
    s = jax.lax.dot_general(q, k, (((1,), (1,)), ((), ())),
                            preferred_element_type=jnp.float32) * scale
    m = jnp.max(s, axis=-1, keepdims=True)
    p = jnp.exp(s - m)
    l = jnp.sum(p, axis=-1, keepdims=True)
    o = jnp.dot(p, v_ref[...], preferred_element_type=jnp.float32)
    o_ref[...] = o / l


def _attention(qkv):
    scale = float(_HD) ** -0.5
    nlt = _L // _TQ
    q_spec = pl.BlockSpec(
        (_TQ, _HD),
        lambda g, qi: ((g // _NKV) * nlt + qi % nlt,
                       (g % _NKV) * _REP + qi // nlt))
    k_spec = pl.BlockSpec((_L, _HD), lambda g, qi: (g // _NKV, _NH + g % _NKV))
    v_spec = pl.BlockSpec((_L, _HD),
                          lambda g, qi: (g // _NKV, _NH + _NKV + g % _NKV))
    return pl.pallas_call(
        functools.partial(_attn_kernel, scale),
        grid=(_B * _NKV, _REP * nlt),
        in_specs=[q_spec, k_spec, v_spec],
        out_specs=pl.BlockSpec(
            (_TQ, _HD),
            lambda g, qi: ((g // _NKV) * nlt + qi % nlt,
                           (g % _NKV) * _REP + qi // nlt)),
        out_shape=jax.ShapeDtypeStruct((_M, _NH * _HD), jnp.float32),
        compiler_params=pltpu.CompilerParams(
            dimension_semantics=("parallel", "arbitrary")),
    )(qkv, qkv, qkv)


def _wo_kernel(a_ref, w_ref, r_ref, o_ref):
    o_ref[...] = r_ref[...] + jnp.dot(a_ref[...], w_ref[...],
                                      preferred_element_type=jnp.float32)


def _wo_residual(attn, wo, res):
    return pl.pallas_call(
        _wo_kernel,
        grid=(_M // _TMW,),
        in_specs=[
            pl.BlockSpec((_TMW, _NH * _HD), lambda i: (i, 0)),
            pl.BlockSpec((_NH * _HD, _D), lambda i: (0, 0)),
            pl.BlockSpec((_TMW, _D), lambda i: (i, 0)),
        ],
        out_specs=pl.BlockSpec((_TMW, _D), lambda i: (i, 0)),
        out_shape=jax.ShapeDtypeStruct((_M, _D), jnp.float32),
        compiler_params=pltpu.CompilerParams(
            dimension_semantics=("parallel",),
            vmem_limit_bytes=56 * 1024 * 1024),
    )(attn, wo, res)


def _ffn_kernel(x_ref, nw_ref, w1_ref, w3_ref, w2_ref, o_ref, xn_ref, acc_ref):
    h = pl.program_id(1)

    @pl.when(h == 0)
    def _():
        xn_ref[...] = _rmsnorm(x_ref[...], nw_ref[...])
        acc_ref[...] = jnp.zeros_like(acc_ref)

    xn = xn_ref[...]
    a = jnp.dot(xn, w1_ref[...], preferred_element_type=jnp.float32)
    b = jnp.dot(xn, w3_ref[...], preferred_element_type=jnp.float32)
    g = (a * jax.lax.logistic(a)) * b
    acc_ref[...] += jnp.dot(g, w2_ref[...],
                            preferred_element_type=jnp.float32)

    @pl.when(h == pl.num_programs(1) - 1)
    def _():
        o_ref[...] = x_ref[...] + acc_ref[...]


def _ffn(x, nw, w1, w3, w2):
    return pl.pallas_call(
        _ffn_kernel,
        grid=(_M // _TMF, _H // _TH),
        in_specs=[
            pl.BlockSpec((_TMF, _D), lambda i, h: (i, 0)),
            pl.BlockSpec((1, _D), lambda i, h: (0, 0)),
            pl.BlockSpec((_D, _TH), lambda i, h: (0, h)),
            pl.BlockSpec((_D, _TH), lambda i, h: (0, h)),
            pl.BlockSpec((_TH, _D), lambda i, h: (h, 0)),
        ],
        out_specs=pl.BlockSpec((_TMF, _D), lambda i, h: (i, 0)),
        out_shape=jax.ShapeDtypeStruct((_M, _D), jnp.float32),
        scratch_shapes=[pltpu.VMEM((_TMF, _D), jnp.float32),
                        pltpu.VMEM((_TMF, _D), jnp.float32)],
        compiler_params=pltpu.CompilerParams(
            dimension_semantics=("parallel", "arbitrary"),
            vmem_limit_bytes=56 * 1024 * 1024),
    )(x, nw.reshape(1, _D), w1, w3, w2)


def _norm_kernel(x_ref, nw_ref, o_ref):
    o_ref[...] = _rmsnorm(x_ref[...], nw_ref[...])


def _final_norm(x, nw):
    return pl.pallas_call(
        _norm_kernel,
        grid=(_M // _TMF,),
        in_specs=[pl.BlockSpec((_TMF, _D), lambda i: (i, 0)),
                  pl.BlockSpec((1, _D), lambda i: (0, 0))],
        out_specs=pl.BlockSpec((_TMF, _D), lambda i: (i, 0)),
        out_shape=jax.ShapeDtypeStruct((_M, _D), jnp.float32),
        compiler_params=pltpu.CompilerParams(
            dimension_semantics=("parallel",)),
    )(x, nw.reshape(1, _D))


def _logits_kernel(xn_ref, w_ref, o_ref):
    o_ref[...] = jnp.dot(xn_ref[...], w_ref[...],
                         preferred_element_type=jnp.float32)


def _logits(xn, w_out):
    return pl.pallas_call(
        _logits_kernel,
        grid=(_V // _TV,),
        in_specs=[pl.BlockSpec((_M, _D), lambda j: (0, 0)),
                  pl.BlockSpec((_D, _TV), lambda j: (0, j))],
        out_specs=pl.BlockSpec((_M, _TV), lambda j: (0, j)),
        out_shape=jax.ShapeDtypeStruct((_M, _V), jnp.float32),
        compiler_params=pltpu.CompilerParams(
            dimension_semantics=("parallel",),
            vmem_limit_bytes=56 * 1024 * 1024),
    )(xn, w_out)


def _rope_tables():
    inv_freq = 1.0 / (_THETA ** (jnp.arange(0, _HD, 2, dtype=jnp.float32) / _HD))
    t = jnp.arange(_L, dtype=jnp.float32)
    freqs = t[:, None] * inv_freq[None, :]
    emb = jnp.concatenate([freqs, freqs], axis=-1)
    cos = jnp.cos(emb)
    sin = jnp.sin(emb)
    sign = jnp.where(jnp.arange(_HD) % 2 == 0, -1.0, 1.0)
    sin2 = sin * sign[None, :]
    cos_t = jnp.tile(cos, (_B, 4))
    sin_t = jnp.tile(sin2, (_B, 4))
    return cos_t, sin_t


def _layer(x, attn_norm, ffn_norm, wq, wk, wv, wo, w1, w3, w2, cos_t, sin_t):
    qkv = _qkv_rope(x, attn_norm, wq, wk, wv, cos_t, sin_t)
    attn = _attention(qkv)
    h = _wo_residual(attn, wo, x)
    return _ffn(h, ffn_norm, w1, w3, w2)


def kernel(tokens, tok_emb, norm_w, w_out,
           l0_attn_norm, l0_ffn_norm, l0_wq, l0_wk, l0_wv, l0_wo,
           l0_w1, l0_w3, l0_w2,
           l1_attn_norm, l1_ffn_norm, l1_wq, l1_wk, l1_wv, l1_wo,
           l1_w1, l1_w3, l1_w2):
    cos_t, sin_t = _rope_tables()
    x = tok_emb[tokens].reshape(_M, _D)
    x = _layer(x, l0_attn_norm, l0_ffn_norm, l0_wq, l0_wk, l0_wv, l0_wo,
               l0_w1, l0_w3, l0_w2, cos_t, sin_t)
    x = _layer(x, l1_attn_norm, l1_ffn_norm, l1_wq, l1_wk, l1_wv, l1_wo,
               l1_w1, l1_w3, l1_w2, cos_t, sin_t)
    xn = _final_norm(x, norm_w)
    logits = _logits(xn, w_out)
    return logits.reshape(_B, _L, _V)


# speedup vs baseline: 2.8111x; 1.1617x over previous
import functools

import jax
import jax.numpy as jnp
from jax.experimental import pallas as pl
from jax.experimental.pallas import tpu as pltpu

_D = 2048
_HD = 128
_NH = 16
_NKV = 4
_REP = _NH // _NKV
_H = 7168
_V = 32000
_B = 4
_L = 512
_M = _B * _L
_EPS = 1e-5
_THETA = 10000.0

_TMQ = 512
_TMF = 512
_TH = 512
_TV = 256
_GW = _REP * _HD


def _rmsnorm(x, nw):
    var = jnp.mean(x * x, axis=-1, keepdims=True)
    return (x * jax.lax.rsqrt(var + _EPS)) * nw


def _rope(y, cos, sin2):
    even = jax.lax.broadcasted_iota(jnp.int32, y.shape, 1) % 2 == 0
    n = y.shape[1]
    swap = jnp.where(even, pltpu.roll(y, n - 1, 1), pltpu.roll(y, 1, 1))
    return y * cos + swap * sin2


def _qkv_kernel(x_ref, nw_ref, wq_ref, wk_ref, wv_ref, q_ref, k_ref, v_ref):
    xn = _rmsnorm(x_ref[...], nw_ref[...])
    q_ref[...] = jnp.dot(xn, wq_ref[...], preferred_element_type=jnp.float32)
    k_ref[...] = jnp.dot(xn, wk_ref[...], preferred_element_type=jnp.float32)
    v_ref[...] = jnp.dot(xn, wv_ref[...], preferred_element_type=jnp.float32)


def _qkv(x, nw, wq, wk, wv):
    kvw = _NKV * _HD
    return pl.pallas_call(
        _qkv_kernel,
        grid=(_M // _TMQ,),
        in_specs=[
            pl.BlockSpec((_TMQ, _D), lambda i: (i, 0)),
            pl.BlockSpec((1, _D), lambda i: (0, 0)),
            pl.BlockSpec((_D, _NH * _HD), lambda i: (0, 0)),
            pl.BlockSpec((_D, kvw), lambda i: (0, 0)),
            pl.BlockSpec((_D, kvw), lambda i: (0, 0)),
        ],
        out_specs=[
            pl.BlockSpec((_TMQ, _NH * _HD), lambda i: (i, 0)),
            pl.BlockSpec((_TMQ, kvw), lambda i: (i, 0)),
            pl.BlockSpec((_TMQ, kvw), lambda i: (i, 0)),
        ],
        out_shape=[
            jax.ShapeDtypeStruct((_M, _NH * _HD), jnp.float32),
            jax.ShapeDtypeStruct((_M, kvw), jnp.float32),
            jax.ShapeDtypeStruct((_M, kvw), jnp.float32),
        ],
        compiler_params=pltpu.CompilerParams(
            dimension_semantics=("parallel",),
            vmem_limit_bytes=56 * 1024 * 1024),
    )(x, nw.reshape(1, _D), wq, wk, wv)


def _attn_wo_kernel(scale, q_ref, k_ref, v_ref, cq_ref, sq_ref, ck_ref,
                    sk_ref, wo_ref, r_ref, o_ref, acc_ref):
    kv = pl.program_id(1)

    @pl.when(kv == 0)
    def _():
        acc_ref[...] = r_ref[...]

    q = _rope(q_ref[...], cq_ref[...], sq_ref[...])
    k = _rope(k_ref[...], ck_ref[...], sk_ref[...])
    v = v_ref[...]
    outs = []
    for r in range(_REP):
        qr = q[:, r * _HD:(r + 1) * _HD]
        s = jax.lax.dot_general(qr, k, (((1,), (1,)), ((), ())),
                                preferred_element_type=jnp.float32) * scale
        m = jnp.max(s, axis=-1, keepdims=True)
        p = jnp.exp(s - m)
        l = jnp.sum(p, axis=-1, keepdims=True)
        outs.append(jnp.dot(p, v, preferred_element_type=jnp.float32) / l)
    o = jnp.concatenate(outs, axis=1)
    acc_ref[...] += jnp.dot(o, wo_ref[...],
                            preferred_element_type=jnp.float32)

    @pl.when(kv == pl.num_programs(1) - 1)
    def _():
        o_ref[...] = acc_ref[...]


def _attn_wo(q, k, v, wo, res, cos_t, sin_t):
    scale = float(_HD) ** -0.5
    return pl.pallas_call(
        functools.partial(_attn_wo_kernel, scale),
        grid=(_B, _NKV),
        in_specs=[
            pl.BlockSpec((_L, _GW), lambda b, kv: (b, kv)),
            pl.BlockSpec((_L, _HD), lambda b, kv: (b, kv)),
            pl.BlockSpec((_L, _HD), lambda b, kv: (b, kv)),
            pl.BlockSpec((_L, _GW), lambda b, kv: (b, 0)),
            pl.BlockSpec((_L, _GW), lambda b, kv: (b, 0)),
            pl.BlockSpec((_L, _HD), lambda b, kv: (b, 0)),
            pl.BlockSpec((_L, _HD), lambda b, kv: (b, 0)),
            pl.BlockSpec((_GW, _D), lambda b, kv: (kv, 0)),
            pl.BlockSpec((_L, _D), lambda b, kv: (b, 0)),
        ],
        out_specs=pl.BlockSpec((_L, _D), lambda b, kv: (b, 0)),
        out_shape=jax.ShapeDtypeStruct((_M, _D), jnp.float32),
        scratch_shapes=[pltpu.VMEM((_L, _D), jnp.float32)],
        compiler_params=pltpu.CompilerParams(
            dimension_semantics=("parallel", "arbitrary"),
            vmem_limit_bytes=56 * 1024 * 1024),
    )(q, k, v, cos_t, sin_t, cos_t, sin_t, wo, res)


def _ffn_kernel(x_ref, nw_ref, w1_ref, w3_ref, w2_ref, o_ref, xn_ref, acc_ref):
    h = pl.program_id(1)

    @pl.when(h == 0)
    def _():
        xn_ref[...] = _rmsnorm(x_ref[...], nw_ref[...])
        acc_ref[...] = jnp.zeros_like(acc_ref)

    xn = xn_ref[...]
    a = jnp.dot(xn, w1_ref[...], preferred_element_type=jnp.float32)
    b = jnp.dot(xn, w3_ref[...], preferred_element_type=jnp.float32)
    g = (a * jax.lax.logistic(a)) * b
    acc_ref[...] += jnp.dot(g, w2_ref[...],
                            preferred_element_type=jnp.float32)

    @pl.when(h == pl.num_programs(1) - 1)
    def _():
        o_ref[...] = x_ref[...] + acc_ref[...]


def _ffn(x, nw, w1, w3, w2):
    return pl.pallas_call(
        _ffn_kernel,
        grid=(_M // _TMF, _H // _TH),
        in_specs=[
            pl.BlockSpec((_TMF, _D), lambda i, h: (i, 0)),
            pl.BlockSpec((1, _D), lambda i, h: (0, 0)),
            pl.BlockSpec((_D, _TH), lambda i, h: (0, h)),
            pl.BlockSpec((_D, _TH), lambda i, h: (0, h)),
            pl.BlockSpec((_TH, _D), lambda i, h: (h, 0)),
        ],
        out_specs=pl.BlockSpec((_TMF, _D), lambda i, h: (i, 0)),
        out_shape=jax.ShapeDtypeStruct((_M, _D), jnp.float32),
        scratch_shapes=[pltpu.VMEM((_TMF, _D), jnp.float32),
                        pltpu.VMEM((_TMF, _D), jnp.float32)],
        compiler_params=pltpu.CompilerParams(
            dimension_semantics=("parallel", "arbitrary"),
            vmem_limit_bytes=56 * 1024 * 1024),
    )(x, nw.reshape(1, _D), w1, w3, w2)


def _logits_kernel(x_ref, nw_ref, w_ref, o_ref, xn_ref):
    j = pl.program_id(0)

    @pl.when(j == 0)
    def _():
        xn_ref[...] = _rmsnorm(x_ref[...], nw_ref[...])

    o_ref[...] = jnp.dot(xn_ref[...], w_ref[...],
                         preferred_element_type=jnp.float32)


def _logits(x, nw, w_out):
    return pl.pallas_call(
        _logits_kernel,
        grid=(_V // _TV,),
        in_specs=[pl.BlockSpec((_M, _D), lambda j: (0, 0)),
                  pl.BlockSpec((1, _D), lambda j: (0, 0)),
                  pl.BlockSpec((_D, _TV), lambda j: (0, j))],
        out_specs=pl.BlockSpec((_M, _TV), lambda j: (0, j)),
        out_shape=jax.ShapeDtypeStruct((_M, _V), jnp.float32),
        scratch_shapes=[pltpu.VMEM((_M, _D), jnp.float32)],
        compiler_params=pltpu.CompilerParams(
            dimension_semantics=("arbitrary",),
            vmem_limit_bytes=56 * 1024 * 1024),
    )(x, nw.reshape(1, _D), w_out)


def _rope_tables():
    inv_freq = 1.0 / (_THETA ** (jnp.arange(0, _HD, 2, dtype=jnp.float32) / _HD))
    t = jnp.arange(_L, dtype=jnp.float32)
    freqs = t[:, None] * inv_freq[None, :]
    emb = jnp.concatenate([freqs, freqs], axis=-1)
    cos = jnp.cos(emb)
    sin = jnp.sin(emb)
    sign = jnp.where(jnp.arange(_HD) % 2 == 0, -1.0, 1.0)
    sin2 = sin * sign[None, :]
    cos_t = jnp.tile(cos, (_B, _REP))
    sin_t = jnp.tile(sin2, (_B, _REP))
    return cos_t, sin_t


def _layer(x, attn_norm, ffn_norm, wq, wk, wv, wo, w1, w3, w2, cos_t, sin_t):
    q, k, v = _qkv(x, attn_norm, wq, wk, wv)
    h = _attn_wo(q, k, v, wo, x, cos_t, sin_t)
    return _ffn(h, ffn_norm, w1, w3, w2)


def kernel(tokens, tok_emb, norm_w, w_out,
           l0_attn_norm, l0_ffn_norm, l0_wq, l0_wk, l0_wv, l0_wo,
           l0_w1, l0_w3, l0_w2,
           l1_attn_norm, l1_ffn_norm, l1_wq, l1_wk, l1_wv, l1_wo,
           l1_w1, l1_w3, l1_w2):
    cos_t, sin_t = _rope_tables()
    x = tok_emb[tokens].reshape(_M, _D)
    x = _layer(x, l0_attn_norm, l0_ffn_norm, l0_wq, l0_wk, l0_wv, l0_wo,
               l0_w1, l0_w3, l0_w2, cos_t, sin_t)
    x = _layer(x, l1_attn_norm, l1_ffn_norm, l1_wq, l1_wk, l1_wv, l1_wo,
               l1_w1, l1_w3, l1_w2, cos_t, sin_t)
    logits = _logits(x, norm_w, w_out)
    return logits.reshape(_B, _L, _V)
